# baseline (speedup 1.0000x reference)
import functools

import jax
import jax.numpy as jnp
from jax import lax
from jax.experimental import pallas as pl
from jax.experimental.pallas import tpu as pltpu
from jax.experimental.pallas import tpu_sc as plsc

VOCAB = 100000
EMB = 300
PAD_D = 16
VBLK = 4000

NUM_CORES = 2
NUM_SUBCORES = 16
NW = NUM_CORES * NUM_SUBCORES
LANES = 16


def _mlp_table_body(tab_ref, w1_ref, b1_ref, w2_ref, b2_ref, out_ref):
    emb = tab_ref[...]
    h = jnp.dot(emb, w1_ref[...], preferred_element_type=jnp.float32)
    h = jnp.maximum(h + b1_ref[...], 0.0)
    out_ref[...] = (
        jnp.dot(h, w2_ref[...], preferred_element_type=jnp.float32) + b2_ref[...]
    )


def _fused_table(table, W1, b1, W2p, b2p):
    grid = VOCAB // VBLK
    return pl.pallas_call(
        _mlp_table_body,
        grid=(grid,),
        in_specs=[
            pl.BlockSpec((VBLK, EMB), lambda i: (i, 0)),
            pl.BlockSpec((EMB, 8), lambda i: (0, 0)),
            pl.BlockSpec((1, 8), lambda i: (0, 0)),
            pl.BlockSpec((8, PAD_D), lambda i: (0, 0)),
            pl.BlockSpec((1, PAD_D), lambda i: (0, 0)),
        ],
        out_specs=pl.BlockSpec((VBLK, PAD_D), lambda i: (i, 0)),
        out_shape=jax.ShapeDtypeStruct((VOCAB, PAD_D), jnp.float32),
    )(table, W1, b1, W2p, b2p)


def _make_gather(n_idx):
    b_per_w = n_idx // NW
    mesh = plsc.VectorSubcoreMesh(core_axis_name="c", subcore_axis_name="s")

    @functools.partial(
        pl.kernel,
        mesh=mesh,
        compiler_params=pltpu.CompilerParams(use_tc_tiling_on_sc=False),
        out_type=jax.ShapeDtypeStruct((n_idx, 2), jnp.float32),
        scratch_types=[
            pltpu.VMEM((b_per_w,), jnp.int32),
            pltpu.VMEM((b_per_w, PAD_D), jnp.float32),
            pltpu.SemaphoreType.DMA,
        ],
    )
    def gather(tab_hbm, idx_hbm, out_hbm, idx_v, rows_v, sem):
        wid = lax.axis_index("s") * NUM_CORES + lax.axis_index("c")
        base = wid * b_per_w
        pltpu.sync_copy(idx_hbm.at[pl.ds(base, b_per_w)], idx_v)
        pltpu.async_copy(tab_hbm.at[idx_v], rows_v, sem).wait()

        pltpu.sync_copy(rows_v.at[:, pl.ds(0, 2)], out_hbm.at[pl.ds(base, b_per_w)])

    return gather


def kernel(x, table, W1, b1, W2, b2):
    B, S = x.shape
    W1p = jnp.zeros((EMB, 8), jnp.float32).at[:, :3].set(W1)
    b1p = jnp.zeros((1, 8), jnp.float32).at[:, :3].set(b1)
    W2p = jnp.zeros((8, PAD_D), jnp.float32).at[:3, :2].set(W2)
    b2p = jnp.zeros((1, PAD_D), jnp.float32).at[:, :2].set(b2)

    fused = _fused_table(table, W1p, b1p, W2p, b2p)

    idx = x.reshape(-1).astype(jnp.int32)
    flat = _make_gather(idx.shape[0])(fused, idx)
    return flat.reshape(B, S, 2)


# speedup vs baseline: 2.8306x; 2.8306x over previous
import functools

import jax
import jax.numpy as jnp
from jax import lax
from jax.experimental import pallas as pl
from jax.experimental.pallas import tpu as pltpu
from jax.experimental.pallas import tpu_sc as plsc

VOCAB = 100000
EMB = 300
PAD_D = 16
VBLK = 4096

NUM_CORES = 2
NUM_SUBCORES = 16
NW = NUM_CORES * NUM_SUBCORES


def _mlp_table_body(tab_ref, w1_ref, b1_ref, w2_ref, b2_ref, out_ref):
    emb_t = tab_ref[...]
    h = lax.dot_general(
        emb_t, w1_ref[...], (((0,), (0,)), ((), ())),
        preferred_element_type=jnp.float32,
    )
    h = jnp.maximum(h + b1_ref[...], 0.0)
    out_ref[...] = (
        jnp.dot(h, w2_ref[...], preferred_element_type=jnp.float32) + b2_ref[...]
    )


def _fused_table(table_t, W1, b1, W2p, b2p):
    grid = pl.cdiv(VOCAB, VBLK)
    return pl.pallas_call(
        _mlp_table_body,
        grid=(grid,),
        in_specs=[
            pl.BlockSpec((EMB, VBLK), lambda i: (0, i)),
            pl.BlockSpec((EMB, 8), lambda i: (0, 0)),
            pl.BlockSpec((1, 8), lambda i: (0, 0)),
            pl.BlockSpec((8, PAD_D), lambda i: (0, 0)),
            pl.BlockSpec((1, PAD_D), lambda i: (0, 0)),
        ],
        out_specs=pl.BlockSpec((VBLK, PAD_D), lambda i: (i, 0)),
        out_shape=jax.ShapeDtypeStruct((VOCAB, PAD_D), jnp.float32),
    )(table_t, W1, b1, W2p, b2p)


def _make_gather(n_idx):
    b_per_w = n_idx // NW
    mesh = plsc.VectorSubcoreMesh(core_axis_name="c", subcore_axis_name="s")

    @functools.partial(
        pl.kernel,
        mesh=mesh,
        compiler_params=pltpu.CompilerParams(use_tc_tiling_on_sc=False),
        out_type=jax.ShapeDtypeStruct((n_idx, PAD_D), jnp.float32),
        scratch_types=[
            pltpu.VMEM((b_per_w,), jnp.int32),
            pltpu.VMEM((b_per_w, PAD_D), jnp.float32),
            pltpu.SemaphoreType.DMA,
        ],
    )
    def gather(tab_hbm, idx_hbm, out_hbm, idx_v, rows_v, sem):
        wid = lax.axis_index("s") * NUM_CORES + lax.axis_index("c")
        base = wid * b_per_w
        pltpu.sync_copy(idx_hbm.at[pl.ds(base, b_per_w)], idx_v)
        pltpu.async_copy(tab_hbm.at[idx_v], rows_v, sem).wait()
        pltpu.sync_copy(rows_v, out_hbm.at[pl.ds(base, b_per_w)])

    return gather


def kernel(x, table, W1, b1, W2, b2):
    B, S = x.shape
    W1p = jnp.zeros((EMB, 8), jnp.float32).at[:, :3].set(W1)
    b1p = jnp.zeros((1, 8), jnp.float32).at[:, :3].set(b1)
    W2p = jnp.zeros((8, PAD_D), jnp.float32).at[:3, :2].set(W2)
    b2p = jnp.zeros((1, PAD_D), jnp.float32).at[:, :2].set(b2)

    fused = _fused_table(table.T, W1p, b1p, W2p, b2p)

    idx = x.reshape(-1).astype(jnp.int32)
    rows = _make_gather(idx.shape[0])(fused, idx)
    return rows[:, :2].reshape(B, S, 2)
